# baseline (speedup 1.0000x reference)
import functools

import jax
import jax.numpy as jnp
from jax import lax
from jax.experimental import pallas as pl
from jax.experimental.pallas import tpu as pltpu
from jax.experimental.pallas import tpu_sc as plsc

B = 16384
F = 26
D = 16
NT = F + 2
CONCAT = NT * D
HID = 256
FIELD_VOCAB = 100000

NC, NS = 2, 16
NW = NC * NS
BPW = B // NW
CH = 128
NCH = BPW // CH


def _gather_body(idx_hbm, user_tbl, item_tbl, feat_tbl, out_hbm,
                 idx_v, rows_v, sem):
    wid = lax.axis_index("s") * NC + lax.axis_index("c")
    base = wid * BPW
    pltpu.sync_copy(idx_hbm.at[:, pl.ds(base, BPW)], idx_v)
    for f in range(NT):
        tbl = user_tbl if f == 0 else (item_tbl if f == 1 else feat_tbl)
        cps = [
            pltpu.async_copy(
                tbl.at[idx_v.at[f, pl.ds(c * CH, CH)]],
                rows_v.at[pl.ds(c * CH, CH)],
                sem,
            )
            for c in range(NCH)
        ]
        for cp in cps:
            cp.wait()
        pltpu.sync_copy(rows_v,
                        out_hbm.at[pl.ds(base, BPW), pl.ds(f * D, D)])


@functools.partial(
    pl.kernel,
    out_type=jax.ShapeDtypeStruct((B, CONCAT), jnp.float32),
    mesh=plsc.VectorSubcoreMesh(core_axis_name="c", subcore_axis_name="s"),
    scratch_types=[
        pltpu.VMEM((NT, BPW), jnp.int32),
        pltpu.VMEM((BPW, D), jnp.float32),
        pltpu.SemaphoreType.DMA,
    ],
    compiler_params=pltpu.CompilerParams(use_tc_tiling_on_sc=False),
)
def _gather_all(idx, user_tbl, item_tbl, feat_tbl, out, idx_v, rows_v, sem):
    _gather_body(idx, user_tbl, item_tbl, feat_tbl, out, idx_v, rows_v, sem)


BLK = 512


def _mlp_body(x_ref, w1t_ref, b1_ref, w2_ref, b2_ref, o_ref):
    x = x_ref[...]
    h = jnp.dot(x, w1t_ref[...], preferred_element_type=jnp.float32)
    h = jnp.maximum(h + b1_ref[...], 0.0)
    d = jnp.sum(h * w2_ref[...], axis=1, keepdims=True)
    fm = jnp.sum(x, axis=1, keepdims=True)
    z = fm + d + b2_ref[...]
    o_ref[...] = 10.0 / (1.0 + jnp.exp(-z))


def _mlp(fm_terms, w1t, b1, w2, b2):
    return pl.pallas_call(
        _mlp_body,
        grid=(B // BLK,),
        in_specs=[
            pl.BlockSpec((BLK, CONCAT), lambda i: (i, 0)),
            pl.BlockSpec((CONCAT, HID), lambda i: (0, 0)),
            pl.BlockSpec((1, HID), lambda i: (0, 0)),
            pl.BlockSpec((1, HID), lambda i: (0, 0)),
            pl.BlockSpec((1, 1), lambda i: (0, 0)),
        ],
        out_specs=pl.BlockSpec((BLK, 1), lambda i: (i, 0)),
        out_shape=jax.ShapeDtypeStruct((B, 1), jnp.float32),
    )(fm_terms, w1t, b1, w2, b2)


def kernel(user, item, feature, user_table, item_table, feat_tables,
           W1, b1, W2, b2):
    offs = jnp.arange(F, dtype=jnp.int32) * FIELD_VOCAB
    idx_all = jnp.concatenate(
        [user[None].astype(jnp.int32),
         item[None].astype(jnp.int32),
         (feature.astype(jnp.int32) + offs[None, :]).T],
        axis=0)
    feat_flat = feat_tables.reshape(F * FIELD_VOCAB, D)
    fm_terms = _gather_all(idx_all, user_table, item_table, feat_flat)
    return _mlp(fm_terms, W1.T, b1.reshape(1, HID), W2.reshape(1, HID),
                b2.reshape(1, 1))
